# initial kernel scaffold (unmeasured)
import jax
import jax.numpy as jnp
from jax import lax
from jax.experimental import pallas as pl
from jax.experimental.pallas import tpu as pltpu

B, H, D = 16, 16, 64
NB, BS = 128, 16
P_LOC = 128
T_LOC = P_LOC * BS
SCALE = D ** -0.5
NEG = -1e30


def kernel(Q, K, V, bt, lens):
    lens2 = lens.reshape(B, 1)

    def body(q_ref, k_ref, v_ref, bt_ref, lens_ref, out_ref,
             send_buf, recv_buf, send_sem, recv_sem):
        my_x = lax.axis_index("x")
        my_y = lax.axis_index("y")
        my_z = lax.axis_index("z")
        partner = (my_x, 1 - my_y, my_z)

        pages = bt_ref[:, :] - my_y * P_LOC
        j_idx = lax.broadcasted_iota(jnp.int32, (B, NB), 1)
        valid = (j_idx < lens_ref[:, :]) & (pages >= 0) & (pages < P_LOC)
        p_iota = lax.broadcasted_iota(jnp.int32, (B, NB, P_LOC), 2)
        onehot = jnp.where((pages[:, :, None] == p_iota) & valid[:, :, None],
                           1.0, 0.0)
        counts = jnp.sum(onehot, axis=1)
        w = jnp.broadcast_to(counts[:, :, None],
                             (B, P_LOC, BS)).reshape(B, T_LOC)

        qb = q_ref[:, :, :, :].reshape(B, H, D).astype(jnp.bfloat16)
        kr = k_ref[:, :, :, :].reshape(T_LOC, H, D).astype(jnp.bfloat16)
        vr = v_ref[:, :, :, :].reshape(T_LOC, H, D).astype(jnp.bfloat16)

        s = lax.dot_general(qb, kr, (((2,), (2,)), ((1,), (1,))),
                            preferred_element_type=jnp.float32)
        s = s * SCALE
        s = jnp.where(w[None, :, :] > 0.0, s, NEG)
        m_loc = jnp.max(s, axis=2)
        p = jnp.exp(s - m_loc[:, :, None]) * w[None, :, :]
        l_loc = jnp.sum(p, axis=2)
        o_loc = lax.dot_general(p.astype(jnp.bfloat16), vr,
                                (((2,), (0,)), ((0,), (1,))),
                                preferred_element_type=jnp.float32)

        send_buf[:, :, :] = jnp.concatenate(
            [m_loc[:, :, None], l_loc[:, :, None], o_loc], axis=2)

        bsem = pltpu.get_barrier_semaphore()
        pl.semaphore_signal(bsem, inc=1, device_id=partner,
                            device_id_type=pl.DeviceIdType.MESH)
        pl.semaphore_wait(bsem, 1)

        rdma = pltpu.make_async_remote_copy(
            src_ref=send_buf, dst_ref=recv_buf,
            send_sem=send_sem, recv_sem=recv_sem,
            device_id=partner, device_id_type=pl.DeviceIdType.MESH)
        rdma.start()
        rdma.wait()

        rb = recv_buf[:, :, :]
        m2 = rb[:, :, 0]
        l2 = rb[:, :, 1]
        o2 = rb[:, :, 2:]
        m_new = jnp.maximum(m_loc, m2)
        a1 = jnp.exp(m_loc - m_new)
        a2 = jnp.exp(m2 - m_new)
        l_new = a1 * l_loc + a2 * l2
        o = (a1[:, :, None] * o_loc + a2[:, :, None] * o2) / l_new[:, :, None]
        out_ref[:, :, :, :] = o.transpose(1, 0, 2).reshape(B, 1, H, D)

    return pl.pallas_call(
        body,
        out_shape=jax.ShapeDtypeStruct((B, 1, H, D), jnp.float32),
        in_specs=[pl.BlockSpec(memory_space=pltpu.VMEM)] * 5,
        out_specs=pl.BlockSpec(memory_space=pltpu.VMEM),
        scratch_shapes=[
            pltpu.VMEM((H, B, D + 2), jnp.float32),
            pltpu.VMEM((H, B, D + 2), jnp.float32),
            pltpu.SemaphoreType.DMA,
            pltpu.SemaphoreType.DMA,
        ],
        compiler_params=pltpu.CompilerParams(
            collective_id=0,
            vmem_limit_bytes=100 * 1024 * 1024,
        ),
    )(Q, K, V, bt, lens2)


# baseline (device time: 91431 ns/iter reference)
import jax
import jax.numpy as jnp
from jax import lax
from jax.experimental import pallas as pl
from jax.experimental.pallas import tpu as pltpu

B, H, D = 16, 16, 64
NB, BS = 128, 16
P_LOC = 128
T_LOC = P_LOC * BS
SCALE = D ** -0.5
NEG = -1e30


def kernel(Q, K, V, bt, lens):
    valid = jnp.arange(NB)[None, :] < lens[:, None]
    onehot = (bt[:, :, None] == jnp.arange(2 * P_LOC)[None, None, :]) & valid[:, :, None]
    counts = onehot.sum(axis=1).astype(jnp.float32)
    w_tok = jnp.repeat(counts, BS, axis=1)
    w_tok = w_tok.reshape(B, 2, T_LOC).transpose(1, 0, 2)

    def body(q_ref, k_ref, v_ref, w_ref, out_ref,
             send_buf, recv_buf, send_sem, recv_sem):
        my_x = lax.axis_index("x")
        my_y = lax.axis_index("y")
        my_z = lax.axis_index("z")
        partner = (my_x, 1 - my_y, my_z)

        w = w_ref[my_y]
        wpos = w > 0.0

        for h in range(H):
            qh = q_ref[:, 0, h, :].astype(jnp.bfloat16)
            kh = k_ref[:, :, h, :].reshape(T_LOC, D).astype(jnp.bfloat16)
            vh = v_ref[:, :, h, :].reshape(T_LOC, D).astype(jnp.bfloat16)

            s = lax.dot_general(qh, kh, (((1,), (1,)), ((), ())),
                                preferred_element_type=jnp.float32)
            s = jnp.where(wpos, s * SCALE, NEG)
            m = jnp.max(s, axis=1, keepdims=True)
            p = jnp.exp(s - m) * w
            l = jnp.sum(p, axis=1, keepdims=True)
            o = lax.dot_general(p.astype(jnp.bfloat16), vh,
                                (((1,), (0,)), ((), ())),
                                preferred_element_type=jnp.float32)
            send_buf[h, :, 0:1] = m
            send_buf[h, :, 1:2] = l
            send_buf[h, :, 2:] = o

        bsem = pltpu.get_barrier_semaphore()
        pl.semaphore_signal(bsem, inc=1, device_id=partner,
                            device_id_type=pl.DeviceIdType.MESH)
        pl.semaphore_wait(bsem, 1)

        rdma = pltpu.make_async_remote_copy(
            src_ref=send_buf, dst_ref=recv_buf,
            send_sem=send_sem, recv_sem=recv_sem,
            device_id=partner, device_id_type=pl.DeviceIdType.MESH)
        rdma.start()
        rdma.wait()

        for h in range(H):
            m1 = send_buf[h, :, 0:1]
            l1 = send_buf[h, :, 1:2]
            o1 = send_buf[h, :, 2:]
            m2 = recv_buf[h, :, 0:1]
            l2 = recv_buf[h, :, 1:2]
            o2 = recv_buf[h, :, 2:]
            mn = jnp.maximum(m1, m2)
            a1 = jnp.exp(m1 - mn)
            a2 = jnp.exp(m2 - mn)
            ln = a1 * l1 + a2 * l2
            out_ref[:, 0, h, :] = (a1 * o1 + a2 * o2) / ln

    return pl.pallas_call(
        body,
        out_shape=jax.ShapeDtypeStruct((B, 1, H, D), jnp.float32),
        in_specs=[pl.BlockSpec(memory_space=pltpu.VMEM)] * 4,
        out_specs=pl.BlockSpec(memory_space=pltpu.VMEM),
        scratch_shapes=[
            pltpu.VMEM((H, B, D + 2), jnp.float32),
            pltpu.VMEM((H, B, D + 2), jnp.float32),
            pltpu.SemaphoreType.DMA,
            pltpu.SemaphoreType.DMA,
        ],
        compiler_params=pltpu.CompilerParams(
            collective_id=0,
            vmem_limit_bytes=100 * 1024 * 1024,
        ),
    )(Q, K, V, w_tok)


# device time: 77580 ns/iter; 1.1785x vs baseline; 1.1785x over previous
import jax
import jax.numpy as jnp
from jax import lax
from jax.experimental import pallas as pl
from jax.experimental.pallas import tpu as pltpu

B, H, D = 16, 16, 64
NB, BS = 128, 16
P_LOC = 128
T_LOC = P_LOC * BS
SCALE = D ** -0.5
NEG = -1e30


def kernel(Q, K, V, bt, lens):
    valid = jnp.arange(NB)[None, :] < lens[:, None]
    onehot = (bt[:, :, None] == jnp.arange(2 * P_LOC)[None, None, :]) & valid[:, :, None]
    counts = onehot.sum(axis=1).astype(jnp.float32)
    w_tok = jnp.repeat(counts, BS, axis=1)
    w_tok = w_tok.reshape(B, 2, T_LOC).transpose(1, 0, 2)

    def body(q_ref, k_ref, v_ref, w_ref, out_ref,
             send_buf, recv_buf, send_sem, recv_sem):
        my_x = lax.axis_index("x")
        my_y = lax.axis_index("y")
        my_z = lax.axis_index("z")
        partner = (my_x, 1 - my_y, my_z)

        w = w_ref[my_y]

        CHUNK = 512
        PP = CHUNK // BS
        NC = T_LOC // CHUNK

        qb = q_ref[:, :, :, :].reshape(B, H, D).astype(jnp.bfloat16)
        m_run = jnp.full((H, B, 1), NEG, dtype=jnp.float32)
        l_run = jnp.zeros((H, B, 1), dtype=jnp.float32)
        o_run = jnp.zeros((H, B, D), dtype=jnp.float32)

        for c in range(NC):
            kc = k_ref[c * PP:(c + 1) * PP].reshape(CHUNK, H, D).astype(jnp.bfloat16)
            vc = v_ref[c * PP:(c + 1) * PP].reshape(CHUNK, H, D).astype(jnp.bfloat16)
            wc = w[:, c * CHUNK:(c + 1) * CHUNK]

            s = lax.dot_general(qb, kc, (((2,), (2,)), ((1,), (1,))),
                                preferred_element_type=jnp.float32)
            s = jnp.where(wc[None] > 0.0, s * SCALE, NEG)
            m_c = jnp.max(s, axis=2, keepdims=True)
            m_new = jnp.maximum(m_run, m_c)
            alpha = jnp.exp(m_run - m_new)
            p = jnp.exp(s - m_new) * wc[None]
            l_run = l_run * alpha + jnp.sum(p, axis=2, keepdims=True)
            o_c = lax.dot_general(p.astype(jnp.bfloat16), vc,
                                  (((2,), (0,)), ((0,), (1,))),
                                  preferred_element_type=jnp.float32)
            o_run = o_run * alpha + o_c
            m_run = m_new

        send_buf[:, :, 0:1] = m_run
        send_buf[:, :, 1:2] = l_run
        send_buf[:, :, 2:] = o_run

        bsem = pltpu.get_barrier_semaphore()
        pl.semaphore_signal(bsem, inc=1, device_id=partner,
                            device_id_type=pl.DeviceIdType.MESH)
        pl.semaphore_wait(bsem, 1)

        rdma = pltpu.make_async_remote_copy(
            src_ref=send_buf, dst_ref=recv_buf,
            send_sem=send_sem, recv_sem=recv_sem,
            device_id=partner, device_id_type=pl.DeviceIdType.MESH)
        rdma.start()
        rdma.wait()

        for h in range(H):
            m1 = send_buf[h, :, 0:1]
            l1 = send_buf[h, :, 1:2]
            o1 = send_buf[h, :, 2:]
            m2 = recv_buf[h, :, 0:1]
            l2 = recv_buf[h, :, 1:2]
            o2 = recv_buf[h, :, 2:]
            mn = jnp.maximum(m1, m2)
            a1 = jnp.exp(m1 - mn)
            a2 = jnp.exp(m2 - mn)
            ln = a1 * l1 + a2 * l2
            out_ref[:, 0, h, :] = (a1 * o1 + a2 * o2) / ln

    return pl.pallas_call(
        body,
        out_shape=jax.ShapeDtypeStruct((B, 1, H, D), jnp.float32),
        in_specs=[pl.BlockSpec(memory_space=pltpu.VMEM)] * 4,
        out_specs=pl.BlockSpec(memory_space=pltpu.VMEM),
        scratch_shapes=[
            pltpu.VMEM((H, B, D + 2), jnp.float32),
            pltpu.VMEM((H, B, D + 2), jnp.float32),
            pltpu.SemaphoreType.DMA,
            pltpu.SemaphoreType.DMA,
        ],
        compiler_params=pltpu.CompilerParams(
            collective_id=0,
            vmem_limit_bytes=100 * 1024 * 1024,
        ),
    )(Q, K, V, w_tok)


# device time: 71262 ns/iter; 1.2830x vs baseline; 1.0887x over previous
import jax
import jax.numpy as jnp
from jax import lax
from jax.experimental import pallas as pl
from jax.experimental.pallas import tpu as pltpu

B, H, D = 16, 16, 64
NB, BS = 128, 16
P_LOC = 128
T_LOC = P_LOC * BS
SCALE = D ** -0.5
NEG = -1e30


def kernel(Q, K, V, bt, lens):
    valid = jnp.arange(NB)[None, :] < lens[:, None]
    onehot = (bt[:, :, None] == jnp.arange(2 * P_LOC)[None, None, :]) & valid[:, :, None]
    counts = onehot.sum(axis=1).astype(jnp.float32)
    w_tok = jnp.repeat(counts, BS, axis=1)
    w_tok = w_tok.reshape(B, 2, T_LOC).transpose(1, 0, 2)

    q2 = Q.reshape(B, H * D)
    k2 = K.reshape(T_LOC, H * D)
    v2 = V.reshape(T_LOC, H * D)

    def body(q_ref, k_ref, v_ref, w_ref, out_ref,
             send_buf, recv_buf, send_sem, recv_sem):
        my_x = lax.axis_index("x")
        my_y = lax.axis_index("y")
        my_z = lax.axis_index("z")
        partner = (my_x, 1 - my_y, my_z)

        w = w_ref[my_y]

        for h in range(H):
            qh = q_ref[:, h * D:(h + 1) * D]
            kh = k_ref[:, h * D:(h + 1) * D]
            vh = v_ref[:, h * D:(h + 1) * D]

            s = lax.dot_general(qh, kh, (((1,), (1,)), ((), ())),
                                preferred_element_type=jnp.float32)
            s = jnp.where(w > 0.0, s * SCALE, NEG)
            m = jnp.max(s, axis=1, keepdims=True)
            p = jnp.exp(s - m) * w
            l = jnp.sum(p, axis=1, keepdims=True)
            o = lax.dot_general(p, vh, (((1,), (0,)), ((), ())),
                                preferred_element_type=jnp.float32)
            send_buf[h, :, 0:1] = m
            send_buf[h, :, 1:2] = l
            send_buf[h, :, 2:] = o

        bsem = pltpu.get_barrier_semaphore()
        pl.semaphore_signal(bsem, inc=1, device_id=partner,
                            device_id_type=pl.DeviceIdType.MESH)
        pl.semaphore_wait(bsem, 1)

        rdma = pltpu.make_async_remote_copy(
            src_ref=send_buf, dst_ref=recv_buf,
            send_sem=send_sem, recv_sem=recv_sem,
            device_id=partner, device_id_type=pl.DeviceIdType.MESH)
        rdma.start()
        rdma.wait()

        for h in range(H):
            m1 = send_buf[h, :, 0:1]
            l1 = send_buf[h, :, 1:2]
            o1 = send_buf[h, :, 2:]
            m2 = recv_buf[h, :, 0:1]
            l2 = recv_buf[h, :, 1:2]
            o2 = recv_buf[h, :, 2:]
            mn = jnp.maximum(m1, m2)
            a1 = jnp.exp(m1 - mn)
            a2 = jnp.exp(m2 - mn)
            ln = a1 * l1 + a2 * l2
            out_ref[:, h * D:(h + 1) * D] = (a1 * o1 + a2 * o2) / ln

    out2 = pl.pallas_call(
        body,
        out_shape=jax.ShapeDtypeStruct((B, H * D), jnp.float32),
        in_specs=[pl.BlockSpec(memory_space=pltpu.VMEM)] * 4,
        out_specs=pl.BlockSpec(memory_space=pltpu.VMEM),
        scratch_shapes=[
            pltpu.VMEM((H, B, D + 2), jnp.float32),
            pltpu.VMEM((H, B, D + 2), jnp.float32),
            pltpu.SemaphoreType.DMA,
            pltpu.SemaphoreType.DMA,
        ],
        compiler_params=pltpu.CompilerParams(
            collective_id=0,
            vmem_limit_bytes=100 * 1024 * 1024,
        ),
    )(q2, k2, v2, w_tok)
    return out2.reshape(B, 1, H, D)


# device time: 55576 ns/iter; 1.6452x vs baseline; 1.2822x over previous
from pathlib import Path

import jax
import jax.numpy as jnp
from jax import lax
from jax.experimental import pallas as pl
from jax.experimental.pallas import tpu as pltpu

try:
    VARIANT = (Path(__file__).parent / "variant.txt").read_text().strip()
except OSError:
    VARIANT = "full"

B, H, D = 16, 16, 64
NB, BS = 128, 16
P_LOC = 128
T_LOC = P_LOC * BS
H_G = 4
SCALE = D ** -0.5
NEG = -1e30


def kernel(Q, K, V, bt, lens):
    valid = jnp.arange(NB)[None, :] < lens[:, None]
    onehot = (bt[:, :, None] == jnp.arange(2 * P_LOC)[None, None, :]) & valid[:, :, None]
    counts = onehot.sum(axis=1).astype(jnp.float32)
    w_tok = jnp.repeat(counts, BS, axis=1)
    w_tok = w_tok.reshape(B, 2, T_LOC).transpose(1, 0, 2)

    qh_all = Q.reshape(B, H, D).transpose(1, 0, 2)

    def body(q_ref, k_hbm, v_hbm, w_ref, out_ref,
             kbuf, vbuf, ybuf_send, ybuf_recv, gbuf,
             ksems, vsems, ysend_sem, yrecv_sem, gsend_sems, grecv_sems):
        my_x = lax.axis_index("x")
        my_y = lax.axis_index("y")
        my_z = lax.axis_index("z")
        partner_y = (my_x, 1 - my_y, my_z)
        my_g = my_x * 2 + my_z
        h0 = my_g * H_G

        w = w_ref[my_y]

        NOSYNC = VARIANT == "nosync"
        if not NOSYNC:
            bsem = pltpu.get_barrier_semaphore()
            pl.semaphore_signal(bsem, inc=1, device_id=partner_y,
                                device_id_type=pl.DeviceIdType.MESH)
            for g in range(4):
                pl.semaphore_signal(bsem, inc=1,
                                    device_id=(g // 2, my_y, g % 2),
                                    device_id_type=pl.DeviceIdType.MESH)
            pl.semaphore_wait(bsem, 5)

        def kv_copy(hh):
            slot = hh % 2
            return (
                pltpu.make_async_copy(k_hbm.at[:, :, h0 + hh, :],
                                      kbuf.at[slot], ksems.at[slot]),
                pltpu.make_async_copy(v_hbm.at[:, :, h0 + hh, :],
                                      vbuf.at[slot], vsems.at[slot]),
            )

        DO_DMA = VARIANT != "nodma"
        if DO_DMA:
            for c in kv_copy(0):
                c.start()

        for hh in range(H_G):
            if DO_DMA:
                if hh + 1 < H_G:
                    for c in kv_copy(hh + 1):
                        c.start()
                for c in kv_copy(hh):
                    c.wait()
            slot = hh % 2

            qh = q_ref[h0 + hh]
            kh = kbuf[slot].reshape(T_LOC, D)
            vh = vbuf[slot].reshape(T_LOC, D)

            s = lax.dot_general(qh, kh, (((1,), (1,)), ((), ())),
                                preferred_element_type=jnp.float32)
            s = jnp.where(w > 0.0, s * SCALE, NEG)
            m = jnp.max(s, axis=1, keepdims=True)
            p = jnp.exp(s - m) * w
            l = jnp.sum(p, axis=1, keepdims=True)
            o = lax.dot_general(p, vh, (((1,), (0,)), ((), ())),
                                preferred_element_type=jnp.float32)
            ybuf_send[hh, :, 0:1] = m
            ybuf_send[hh, :, 1:2] = l
            ybuf_send[hh, :, 2:] = o

        if not NOSYNC:
            yrdma = pltpu.make_async_remote_copy(
                src_ref=ybuf_send, dst_ref=ybuf_recv,
                send_sem=ysend_sem, recv_sem=yrecv_sem,
                device_id=partner_y, device_id_type=pl.DeviceIdType.MESH)
            yrdma.start()
            yrdma.wait()
        else:
            ybuf_recv[...] = ybuf_send[...]

        for hh in range(H_G):
            m1 = ybuf_send[hh, :, 0:1]
            l1 = ybuf_send[hh, :, 1:2]
            o1 = ybuf_send[hh, :, 2:]
            m2 = ybuf_recv[hh, :, 0:1]
            l2 = ybuf_recv[hh, :, 1:2]
            o2 = ybuf_recv[hh, :, 2:]
            mn = jnp.maximum(m1, m2)
            a1 = jnp.exp(m1 - mn)
            a2 = jnp.exp(m2 - mn)
            ln = a1 * l1 + a2 * l2
            fin = (a1 * o1 + a2 * o2) / ln
            gbuf[hh] = fin
            out_ref[h0 + hh] = fin

        for g in range(4 if not NOSYNC else 0):
            @pl.when(g != my_g)
            def _():
                grdma = pltpu.make_async_remote_copy(
                    src_ref=gbuf,
                    dst_ref=out_ref.at[pl.ds(h0, H_G)],
                    send_sem=gsend_sems.at[g],
                    recv_sem=grecv_sems.at[my_g],
                    device_id=(g // 2, my_y, g % 2),
                    device_id_type=pl.DeviceIdType.MESH)
                grdma.start()
                grdma.wait_send()

        for g in range(4 if not NOSYNC else 0):
            @pl.when(g != my_g)
            def _():
                grdma = pltpu.make_async_remote_copy(
                    src_ref=gbuf,
                    dst_ref=out_ref.at[pl.ds(g * H_G, H_G)],
                    send_sem=gsend_sems.at[g],
                    recv_sem=grecv_sems.at[g],
                    device_id=(g // 2, my_y, g % 2),
                    device_id_type=pl.DeviceIdType.MESH)
                grdma.wait_recv()

    out = pl.pallas_call(
        body,
        out_shape=jax.ShapeDtypeStruct((H, B, D), jnp.float32),
        in_specs=[
            pl.BlockSpec(memory_space=pltpu.VMEM),
            pl.BlockSpec(memory_space=pl.ANY),
            pl.BlockSpec(memory_space=pl.ANY),
            pl.BlockSpec(memory_space=pltpu.VMEM),
        ],
        out_specs=pl.BlockSpec(memory_space=pltpu.VMEM),
        scratch_shapes=[
            pltpu.VMEM((2, P_LOC, BS, D), jnp.float32),
            pltpu.VMEM((2, P_LOC, BS, D), jnp.float32),
            pltpu.VMEM((H_G, B, D + 2), jnp.float32),
            pltpu.VMEM((H_G, B, D + 2), jnp.float32),
            pltpu.VMEM((H_G, B, D), jnp.float32),
            pltpu.SemaphoreType.DMA((2,)),
            pltpu.SemaphoreType.DMA((2,)),
            pltpu.SemaphoreType.DMA,
            pltpu.SemaphoreType.DMA,
            pltpu.SemaphoreType.DMA((4,)),
            pltpu.SemaphoreType.DMA((4,)),
        ],
        compiler_params=pltpu.CompilerParams(
            collective_id=0 if VARIANT != "nosync" else None,
            vmem_limit_bytes=100 * 1024 * 1024,
        ),
    )(qh_all, K, V, w_tok)
    return out.transpose(1, 0, 2).reshape(B, 1, H, D)
